# Initial kernel scaffold; baseline (speedup 1.0000x reference)
import jax
import jax.numpy as jnp
from jax.experimental import pallas as pl


def kernel(x, z, edge_index, batch, params):
    raise NotImplementedError("write your pallas kernel here")


# speedup vs baseline: 3.7128x; 3.7128x over previous
import functools

import jax
import jax.numpy as jnp
from jax import lax
from jax.experimental import pallas as pl
from jax.experimental.pallas import tpu as pltpu
from jax.experimental.pallas import tpu_sc as plsc

_N = 10000
_E = 320000
_D = 128
_Z = 64
_H = 128
_G = 64

_NC = 2
_NS = 16
_K = 80
_EC = _E // _NC
_ES = _EC // _NS
_NCHUNK = _ES // _K
_RPS = 632
_NP = _RPS * _NS

_BLK = 1000


def _sc_agg_body(p_hbm, src_hbm, dst_hbm, zrow_hbm, outp_hbm,
                 srcv, dstv, rows, sem, accp):
    c = lax.axis_index("c")
    s = lax.axis_index("s")

    r0 = pl.multiple_of(s * _RPS, 8)
    chunks = [(j * _K, _K) for j in range(_RPS // _K)]
    if _RPS % _K:
        chunks.append(((_RPS // _K) * _K, _RPS % _K))

    pltpu.sync_copy(zrow_hbm, rows)
    for o, sz in chunks:
        pltpu.sync_copy(rows.at[pl.ds(0, sz)], accp.at[pl.ds(r0 + o, sz)])
    plsc.subcore_barrier()

    base = c * _EC + s * _ES

    @pl.loop(0, _NCHUNK)
    def _edge_chunk(i):
        off = pl.multiple_of(base + i * _K, 8)
        pltpu.sync_copy(src_hbm.at[pl.ds(off, _K)], srcv)
        pltpu.sync_copy(dst_hbm.at[pl.ds(off, _K)], dstv)
        pltpu.async_copy(p_hbm.at[srcv], rows, sem).wait()
        pltpu.sync_copy(rows, accp.at[dstv], add=True)

    plsc.subcore_barrier()
    for o, sz in chunks:
        pltpu.sync_copy(accp.at[pl.ds(r0 + o, sz)], rows.at[pl.ds(0, sz)])
        pltpu.sync_copy(rows.at[pl.ds(0, sz)],
                        outp_hbm.at[c].at[pl.ds(r0 + o, sz)])


@functools.lru_cache(maxsize=None)
def _make_sc_agg(width):
    mesh = plsc.VectorSubcoreMesh(core_axis_name="c", subcore_axis_name="s",
                                  num_cores=_NC, num_subcores=_NS)
    scratch = [
        pltpu.VMEM((_K,), jnp.int32),
        pltpu.VMEM((_K,), jnp.int32),
        pltpu.VMEM((_K, width), jnp.float32),
        pltpu.SemaphoreType.DMA,
        pltpu.VMEM_SHARED((_NP, width), jnp.float32),
    ]
    return pl.kernel(
        _sc_agg_body,
        out_type=jax.ShapeDtypeStruct((_NC, _NP, width), jnp.float32),
        mesh=mesh,
        scratch_types=scratch,
        name=f"sc_edge_agg_w{width}",
    )


def _sc_agg(p, src, dst):
    width = p.shape[1]
    zrow = jnp.zeros((_K, width), jnp.float32)
    return _make_sc_agg(width)(p, src, dst, zrow)


def _prep_body(z_ref, b_ref, zoh_ref, cnt_ref):
    zoh_ref[...] = (z_ref[...] == lax.broadcasted_iota(
        jnp.int32, (_BLK, _H), 1)).astype(jnp.float32)
    boh = (b_ref[...] == lax.broadcasted_iota(jnp.int32, (_BLK, _G), 1)
           ).astype(jnp.float32)

    @pl.when(pl.program_id(0) == 0)
    def _():
        cnt_ref[...] = jnp.zeros_like(cnt_ref)

    cnt_ref[...] += jnp.broadcast_to(jnp.sum(boh, axis=0)[:, None], (_G, _H))


def _stats_tail(m, m_ref, stats_ref):
    m_ref[...] = m

    @pl.when(pl.program_id(0) == 0)
    def _():
        stats_ref[...] = jnp.zeros_like(stats_ref)

    stats_ref[...] += jnp.concatenate(
        [jnp.sum(m, axis=0, keepdims=True),
         jnp.sum(m * m, axis=0, keepdims=True),
         jnp.zeros((6, _H), jnp.float32)], axis=0)


def _layer0_body(zoh_ref, x_ref, gza_ref, gzb_ref, gxa_ref, gxb_ref,
                 w1_ref, b1_ref, w2_ref, b2_ref, m_ref, stats_ref):
    uz = (zoh_ref[...] + gza_ref[...] + gzb_ref[...])[:, 0:_Z]
    ux = x_ref[...] + gxa_ref[...] + gxb_ref[...]
    u = jnp.concatenate([uz, uz, ux], axis=1)
    t = jnp.dot(u, w1_ref[...], preferred_element_type=jnp.float32) + b1_ref[...]
    r = jnp.maximum(t, 0.0)
    m = jnp.maximum(jnp.dot(r, w2_ref[...],
                            preferred_element_type=jnp.float32)
                    + b2_ref[...], 0.0)
    _stats_tail(m, m_ref, stats_ref)


def _layer_body(h_ref, ga_ref, gb_ref, w1_ref, b1_ref, w2_ref, b2_ref,
                m_ref, stats_ref):
    u = h_ref[...] + ga_ref[...] + gb_ref[...]
    t = jnp.dot(u, w1_ref[...], preferred_element_type=jnp.float32) + b1_ref[...]
    r = jnp.maximum(t, 0.0)
    m = jnp.maximum(jnp.dot(r, w2_ref[...],
                            preferred_element_type=jnp.float32)
                    + b2_ref[...], 0.0)
    _stats_tail(m, m_ref, stats_ref)


def _bn_body(m_ref, mv_ref, g_ref, be_ref, b_ref, h_ref, seg_ref):
    h = (g_ref[...] * (m_ref[...] - mv_ref[0:1, :])
         * lax.rsqrt(mv_ref[1:2, :] + 1e-5) + be_ref[...])
    h_ref[...] = h
    boh = (b_ref[...] == lax.broadcasted_iota(jnp.int32, (_BLK, _G), 1)
           ).astype(jnp.float32)

    @pl.when(pl.program_id(0) == 0)
    def _():
        seg_ref[...] = jnp.zeros_like(seg_ref)

    seg_ref[...] += lax.dot_general(boh, h, (((0,), (0,)), ((), ())),
                                    preferred_element_type=jnp.float32)


def _final_body(s0_ref, s1_ref, s2_ref, cnt_ref, w1_ref, b1_ref,
                w2_ref, b2_ref, o_ref):
    denom = jnp.maximum(cnt_ref[...], 1.0)
    pooled = jnp.concatenate(
        [s0_ref[...] / denom, s1_ref[...] / denom, s2_ref[...] / denom],
        axis=1)
    q = jnp.maximum(jnp.dot(pooled, w1_ref[...],
                            preferred_element_type=jnp.float32)
                    + b1_ref[...], 0.0)
    s = jnp.sum(q * w2_ref[...], axis=1, keepdims=True) + b2_ref[0:1, 0:1]
    o_ref[...] = jnp.broadcast_to(s, (_G, _H))


def _row_spec(width):
    return pl.BlockSpec((_BLK, width), lambda i: (i, 0))


def _full_spec(shape):
    return pl.BlockSpec(shape, lambda i: tuple(0 for _ in shape))


_GRID = _N // _BLK

_prep = pl.pallas_call(
    _prep_body,
    grid=(_GRID,),
    in_specs=[_row_spec(1), _row_spec(1)],
    out_specs=[_row_spec(_H), _full_spec((_G, _H))],
    out_shape=[jax.ShapeDtypeStruct((_N, _H), jnp.float32),
               jax.ShapeDtypeStruct((_G, _H), jnp.float32)],
)

_layer0 = pl.pallas_call(
    _layer0_body,
    grid=(_GRID,),
    in_specs=[_row_spec(_H), _row_spec(_D),
              _row_spec(_H), _row_spec(_H), _row_spec(_D), _row_spec(_D),
              _full_spec((2 * _Z + _D, _H)), _full_spec((1, _H)),
              _full_spec((_H, _H)), _full_spec((1, _H))],
    out_specs=[_row_spec(_H), _full_spec((8, _H))],
    out_shape=[jax.ShapeDtypeStruct((_N, _H), jnp.float32),
               jax.ShapeDtypeStruct((8, _H), jnp.float32)],
)

_layer = pl.pallas_call(
    _layer_body,
    grid=(_GRID,),
    in_specs=[_row_spec(_H), _row_spec(_H), _row_spec(_H),
              _full_spec((_H, _H)), _full_spec((1, _H)),
              _full_spec((_H, _H)), _full_spec((1, _H))],
    out_specs=[_row_spec(_H), _full_spec((8, _H))],
    out_shape=[jax.ShapeDtypeStruct((_N, _H), jnp.float32),
               jax.ShapeDtypeStruct((8, _H), jnp.float32)],
)

_bn = pl.pallas_call(
    _bn_body,
    grid=(_GRID,),
    in_specs=[_row_spec(_H), _full_spec((8, _H)), _full_spec((1, _H)),
              _full_spec((1, _H)), _row_spec(1)],
    out_specs=[_row_spec(_H), _full_spec((_G, _H))],
    out_shape=[jax.ShapeDtypeStruct((_N, _H), jnp.float32),
               jax.ShapeDtypeStruct((_G, _H), jnp.float32)],
)

_final = pl.pallas_call(
    _final_body,
    grid=(1,),
    in_specs=[_full_spec((_G, _H)), _full_spec((_G, _H)), _full_spec((_G, _H)),
              _full_spec((_G, _H)), _full_spec((3 * _H, _H)),
              _full_spec((1, _H)), _full_spec((1, _H)), _full_spec((1, _H))],
    out_specs=[_full_spec((_G, _H))],
    out_shape=[jax.ShapeDtypeStruct((_G, _H), jnp.float32)],
)


def kernel(x, z, edge_index, batch, params):
    src = edge_index[0].astype(jnp.int32)
    dst = edge_index[1].astype(jnp.int32)
    z2 = z.astype(jnp.int32).reshape(_N, 1)
    b2 = batch.astype(jnp.int32).reshape(_N, 1)

    zoh, cnt = _prep(z2, b2)

    segs = []
    m = stats = None
    for l in range(3):
        if l == 0:
            az = _sc_agg(zoh, src, dst)
            ax = _sc_agg(x, src, dst)
            m, stats = _layer0(
                zoh, x, az[0, :_N], az[1, :_N], ax[0, :_N], ax[1, :_N],
                params["W0a"], params["b0a"].reshape(1, _H),
                params["W0b"], params["b0b"].reshape(1, _H))
        else:
            ah = _sc_agg(h, src, dst)
            m, stats = _layer(
                h, ah[0, :_N], ah[1, :_N],
                params[f"W{l}a"], params[f"b{l}a"].reshape(1, _H),
                params[f"W{l}b"], params[f"b{l}b"].reshape(1, _H))
        mu = stats[0] / _N
        var = stats[1] / _N - mu * mu
        mv = jnp.zeros((8, _H), jnp.float32).at[0].set(mu).at[1].set(var)
        h, seg = _bn(m, mv, params[f"gamma{l}"].reshape(1, _H),
                     params[f"beta{l}"].reshape(1, _H), b2)
        segs.append(seg)

    w2r = params["W_lin2"].reshape(1, _H)
    b2f = jnp.broadcast_to(params["b_lin2"].reshape(1, 1), (1, _H))
    (o128,) = _final(segs[0], segs[1], segs[2], cnt,
                     params["W_lin1"], params["b_lin1"].reshape(1, _H),
                     w2r, b2f)
    return o128[:, 0:1]
